# baseline (speedup 1.0000x reference)
import functools

import jax
import jax.numpy as jnp
from jax.experimental import pallas as pl
from jax.experimental.pallas import tpu as pltpu


def _round_up(x, m):
    return (x + m - 1) // m * m


def _chunk_to_slot(table_ref, base_ref, sh_ref, r):
    base = pl.multiple_of(base_ref[r], 8)
    chunk = table_ref[pl.ds(base, 8), :]
    return pltpu.roll(chunk, sh_ref[r], axis=0)


def _gather_mul_body(cb_ref, cs_ref, lb_ref, ls_ref, table_ref, o_ref, *,
                     groups, rows_per_group):
    Dc = o_ref.shape[1]
    masks = [
        jax.lax.broadcasted_iota(jnp.int32, (8, Dc), 0) == j for j in range(1, 8)
    ]

    def group(g, carry):
        gbase = pl.multiple_of(g * rows_per_group, 8)
        for k in range(rows_per_group // 8):
            obase = pl.multiple_of(gbase + k * 8, 8)
            acc = None
            for j in range(8):
                r = obase + j
                prod = (
                    _chunk_to_slot(table_ref, cb_ref, cs_ref, r)
                    * _chunk_to_slot(table_ref, lb_ref, ls_ref, r)
                )
                acc = prod if acc is None else jnp.where(masks[j - 1], prod, acc)
            o_ref[pl.ds(obase, 8), :] = acc
        return carry

    jax.lax.fori_loop(0, groups, group, 0)


def _gather_body(cb_ref, cs_ref, table_ref, o_ref, *, groups, rows_per_group):
    Dc = o_ref.shape[1]
    masks = [
        jax.lax.broadcasted_iota(jnp.int32, (8, Dc), 0) == j for j in range(1, 8)
    ]

    def group(g, carry):
        gbase = pl.multiple_of(g * rows_per_group, 8)
        for k in range(rows_per_group // 8):
            obase = pl.multiple_of(gbase + k * 8, 8)
            acc = None
            for j in range(8):
                row = _chunk_to_slot(table_ref, cb_ref, cs_ref, obase + j)
                acc = row if acc is None else jnp.where(masks[j - 1], row, acc)
            o_ref[pl.ds(obase, 8), :] = acc
        return carry

    jax.lax.fori_loop(0, groups, group, 0)


def kernel(emb_table, current_ids, last_ids=None):
    V, D = emb_table.shape
    B = current_ids.shape[0]

    NC = 2 if (D % 256 == 0) else 1
    Dc = D // NC

    ROWS_PER_GROUP = 128
    B_pad = _round_up(max(B, 1), ROWS_PER_GROUP)
    groups = B_pad // ROWS_PER_GROUP
    pos8 = jnp.arange(B_pad, dtype=jnp.int32) & 7

    def prep_ids(ids):
        ids = jnp.clip(ids.astype(jnp.int32), 0, V - 1)
        ids = jnp.pad(ids, (0, B_pad - B))
        return ids & ~7, (pos8 - ids) & 7

    table_spec = pl.BlockSpec((V, Dc), lambda i, *_: (0, i))
    out_spec = pl.BlockSpec((B_pad, Dc), lambda i, *_: (0, i))
    out_shape = jax.ShapeDtypeStruct((B_pad, D), emb_table.dtype)

    itemsize = jnp.dtype(emb_table.dtype).itemsize
    n_ids = 1 if last_ids is None else 2
    cost = pl.CostEstimate(
        flops=n_ids * B_pad * D,
        transcendentals=0,
        bytes_accessed=V * D * itemsize + B_pad * D * itemsize + n_ids * B_pad * 4,
    )
    compiler_params = pltpu.CompilerParams(dimension_semantics=("parallel",))

    if last_ids is None:
        body = functools.partial(
            _gather_body, groups=groups, rows_per_group=ROWS_PER_GROUP
        )
        cb, cs = prep_ids(current_ids)
        out = pl.pallas_call(
            body,
            out_shape=out_shape,
            grid_spec=pltpu.PrefetchScalarGridSpec(
                num_scalar_prefetch=2,
                grid=(NC,),
                in_specs=[table_spec],
                out_specs=out_spec,
            ),
            compiler_params=compiler_params,
            cost_estimate=cost,
        )(cb, cs, emb_table)
    else:
        body = functools.partial(
            _gather_mul_body, groups=groups, rows_per_group=ROWS_PER_GROUP
        )
        cb, cs = prep_ids(current_ids)
        lb, ls = prep_ids(last_ids)
        out = pl.pallas_call(
            body,
            out_shape=out_shape,
            grid_spec=pltpu.PrefetchScalarGridSpec(
                num_scalar_prefetch=4,
                grid=(NC,),
                in_specs=[table_spec],
                out_specs=out_spec,
            ),
            compiler_params=compiler_params,
            cost_estimate=cost,
        )(cb, cs, lb, ls, emb_table)

    return out[:B]
